# Initial kernel scaffold; baseline (speedup 1.0000x reference)
import jax
import jax.numpy as jnp
from jax.experimental import pallas as pl


def kernel(x, edge_index, enc_W, enc_b, norm_gamma, norm_beta, linl_W, linl_b, linr_W, last_gamma, last_beta, pred_W, pred_b):
    raise NotImplementedError("write your pallas kernel here")


# speedup vs baseline: 5.3231x; 5.3231x over previous
import functools

import jax
import jax.numpy as jnp
from jax import lax
from jax.experimental import pallas as pl
from jax.experimental.pallas import tpu as pltpu
from jax.experimental.pallas import tpu_sc as plsc

N = 10000
E = 320000
D_IN = 128
HID = 128
HG = 64
OUT = 47
LAYERS = 4

NTILE = 16
NCORE = 2
NW = NTILE * NCORE
NPAD = 10112
STRIPE = NPAD // NTILE
CH = 128
NCHUNK = 80
EPW = CH * NCHUNK
EPAD = NW * EPW
NITER = NCHUNK // 2

@functools.cache
def _sc_mesh():
  return plsc.VectorSubcoreMesh(
      core_axis_name="c", subcore_axis_name="s", num_cores=NCORE,
      num_subcores=NTILE)


@functools.cache
def _make_sc_agg(D):
  @functools.partial(
      pl.kernel,
      out_type=jax.ShapeDtypeStruct((NCORE, NPAD, D), jnp.float32),
      mesh=_sc_mesh(),
      scratch_types=[
          pltpu.VMEM((NCHUNK, CH), jnp.int32),
          pltpu.VMEM((NCHUNK, CH), jnp.int32),
          pltpu.VMEM((CH, D), jnp.float32),
          pltpu.VMEM((CH, D), jnp.float32),
          pltpu.VMEM_SHARED((NPAD, D), jnp.float32),
          pltpu.SemaphoreType.DMA,
          pltpu.SemaphoreType.DMA,
      ],
      compiler_params=pltpu.CompilerParams(use_tc_tiling_on_sc=False),
  )
  def agg(table, src2d, dst2d, zeros, out, src_v, dst_v, rowa, rowb, acc,
          sema, semb):
    c = lax.axis_index("c")
    s = lax.axis_index("s")
    w = c * NTILE + s
    pltpu.sync_copy(zeros.at[pl.ds(s * STRIPE, STRIPE)],
                    acc.at[pl.ds(s * STRIPE, STRIPE)])
    pltpu.sync_copy(src2d.at[w], src_v)
    pltpu.sync_copy(dst2d.at[w], dst_v)
    pltpu.async_copy(table.at[src_v.at[0]], rowa, sema)
    plsc.subcore_barrier()

    def body(g, carry):
      i0 = 2 * g
      i1 = i0 + 1
      pltpu.make_async_copy(table.at[src_v.at[i0]], rowa, sema).wait()
      pltpu.async_copy(table.at[src_v.at[i1]], rowb, semb)
      pltpu.sync_copy(rowa, acc.at[dst_v.at[i0]], add=True)
      pltpu.make_async_copy(table.at[src_v.at[i1]], rowb, semb).wait()

      @pl.when(g < NITER - 1)
      def _():
        pltpu.async_copy(table.at[src_v.at[i0 + 2]], rowa, sema)

      pltpu.sync_copy(rowb, acc.at[dst_v.at[i1]], add=True)
      return carry

    lax.fori_loop(0, NITER, body, 0)
    plsc.subcore_barrier()
    pltpu.sync_copy(acc.at[pl.ds(s * STRIPE, STRIPE)],
                    out.at[c].at[pl.ds(s * STRIPE, STRIPE)])

  return agg


CD = 16


@functools.cache
def _make_sc_count():

  @functools.partial(
      pl.kernel,
      out_type=jax.ShapeDtypeStruct((NCORE, NPAD, CD), jnp.float32),
      mesh=_sc_mesh(),
      scratch_types=[
          pltpu.VMEM((NCHUNK, CH), jnp.int32),
          pltpu.VMEM((CH, CD), jnp.float32),
          pltpu.VMEM_SHARED((NPAD, CD), jnp.float32),
      ],
      compiler_params=pltpu.CompilerParams(use_tc_tiling_on_sc=False),
  )
  def count(dst2d, ones, zeros, out, dst_v, ones_v, acc):
    c = lax.axis_index("c")
    s = lax.axis_index("s")
    w = c * NTILE + s
    pltpu.sync_copy(zeros.at[pl.ds(s * STRIPE, STRIPE)],
                    acc.at[pl.ds(s * STRIPE, STRIPE)])
    pltpu.sync_copy(dst2d.at[w], dst_v)
    pltpu.sync_copy(ones, ones_v)
    plsc.subcore_barrier()

    def body(i, carry):
      pltpu.sync_copy(ones_v, acc.at[dst_v.at[i]], add=True)
      return carry

    lax.fori_loop(0, NCHUNK, body, 0)
    plsc.subcore_barrier()
    pltpu.sync_copy(acc.at[pl.ds(s * STRIPE, STRIPE)],
                    out.at[c].at[pl.ds(s * STRIPE, STRIPE)])

  return count


R = 1000
GRID = N // R


def _ln_relu(x, g, b):
  mu = jnp.mean(x, axis=-1, keepdims=True)
  var = jnp.mean((x - mu) * (x - mu), axis=-1, keepdims=True)
  return jnp.maximum((x - mu) * lax.rsqrt(var + 1e-5) * g + b, 0.0)


def _enc_body(x_ref, w_ref, b_ref, g_ref, be_ref, x0_ref, x1_ref, a_ref):
  h = jnp.dot(x_ref[...], w_ref[...], preferred_element_type=jnp.float32)
  h = h + b_ref[...]
  x0_ref[...] = h[:, :HG]
  x1 = h[:, HG:]
  x1_ref[...] = x1
  a_ref[...] = _ln_relu(x1, g_ref[...], be_ref[...])


_full = lambda shape: pl.BlockSpec(shape, lambda i: (0,) * len(shape))
_rows = lambda shape: pl.BlockSpec(shape, lambda i: (i,) + (0,) * (len(shape) - 1))

_enc = pl.pallas_call(
    _enc_body,
    grid=(GRID,),
    in_specs=[_rows((R, D_IN)), _full((D_IN, HID)), _full((1, HID)),
              _full((1, HG)), _full((1, HG))],
    out_specs=[_rows((R, HG)), _rows((R, HG)), _rows((R, HG))],
    out_shape=[jax.ShapeDtypeStruct((N, HG), jnp.float32)] * 3,
)


def _mean_agg(a0, a1, c0, c1):
  cnt = jnp.maximum(c0[:, :1] + c1[:, :1], 1.0)
  return (a0 + a1) * (1.0 / cnt)


def _post_pre_body(y_ref, o_ref, a0_ref, a1_ref, c0_ref, c1_ref, wl_ref,
                   bl_ref, wr_ref, g_ref, b_ref, yo_ref, oo_ref):
  mean = _mean_agg(a0_ref[...], a1_ref[...], c0_ref[...], c1_ref[...])
  y = (y_ref[...] + bl_ref[...]
       + jnp.dot(mean, wl_ref[...], preferred_element_type=jnp.float32)
       + jnp.dot(o_ref[...], wr_ref[...], preferred_element_type=jnp.float32))
  yo_ref[...] = y
  oo_ref[...] = _ln_relu(y, g_ref[...], b_ref[...])


_post_pre = pl.pallas_call(
    _post_pre_body,
    grid=(GRID,),
    in_specs=[_rows((R, HG)), _rows((R, HG)), _rows((R, HG)), _rows((R, HG)),
              _rows((R, CD)), _rows((R, CD)), _full((HG, HG)), _full((1, HG)),
              _full((HG, HG)), _full((1, HG)), _full((1, HG))],
    out_specs=[_rows((R, HG)), _rows((R, HG))],
    out_shape=[jax.ShapeDtypeStruct((N, HG), jnp.float32)] * 2,
)


def _final_body(y_ref, o_ref, a0_ref, a1_ref, c0_ref, c1_ref, wl_ref, bl_ref,
                wr_ref, x0_ref, lg_ref, lb_ref, pw_ref, pb_ref, out_ref):
  mean = _mean_agg(a0_ref[...], a1_ref[...], c0_ref[...], c1_ref[...])
  y1 = (y_ref[...] + bl_ref[...]
        + jnp.dot(mean, wl_ref[...], preferred_element_type=jnp.float32)
        + jnp.dot(o_ref[...], wr_ref[...], preferred_element_type=jnp.float32))
  h = jnp.concatenate([x0_ref[...], y1], axis=-1)
  hn = _ln_relu(h, lg_ref[...], lb_ref[...])
  logits = jnp.dot(hn, pw_ref[...], preferred_element_type=jnp.float32)
  logits = logits + pb_ref[...]
  m = jnp.max(logits, axis=-1, keepdims=True)
  e = logits - m
  out_ref[...] = e - jnp.log(jnp.sum(jnp.exp(e), axis=-1, keepdims=True))


_final = pl.pallas_call(
    _final_body,
    grid=(GRID,),
    in_specs=[_rows((R, HG)), _rows((R, HG)), _rows((R, HG)), _rows((R, HG)),
              _rows((R, CD)), _rows((R, CD)), _full((HG, HG)), _full((1, HG)),
              _full((HG, HG)), _rows((R, HG)), _full((1, HID)),
              _full((1, HID)), _full((HID, OUT)), _full((1, OUT))],
    out_specs=_rows((R, OUT)),
    out_shape=jax.ShapeDtypeStruct((N, OUT), jnp.float32),
)


def kernel(x, edge_index, enc_W, enc_b, norm_gamma, norm_beta, linl_W, linl_b,
           linr_W, last_gamma, last_beta, pred_W, pred_b):
  src = edge_index[0]
  dst = edge_index[1]
  pad = EPAD - E
  src2d = jnp.concatenate(
      [src, jnp.zeros((pad,), jnp.int32)]).reshape(NW, NCHUNK, CH)
  dst2d = jnp.concatenate(
      [dst, jnp.full((pad,), N, jnp.int32)]).reshape(NW, NCHUNK, CH)
  zeros64 = jnp.zeros((NPAD, HG), jnp.float32)
  zeros16 = jnp.zeros((NPAD, CD), jnp.float32)
  ones16 = jnp.ones((CH, CD), jnp.float32)

  cntp = _make_sc_count()(dst2d, ones16, zeros16)
  c0 = cntp[0]
  c1 = cntp[1]

  r1 = lambda v: v.reshape(1, -1)
  x0, x1, o = _enc(x, enc_W, r1(enc_b), r1(norm_gamma[0, 0]),
                   r1(norm_beta[0, 0]))
  res = [x0, x1]
  for l in range(LAYERS):
    for g in range(2):
      aggp = _make_sc_agg(HG)(o, src2d, dst2d, zeros64)
      if (l, g) == (LAYERS - 1, 1):
        out = _final(res[1], o, aggp[0], aggp[1], c0, c1, linl_W[l, g],
                     r1(linl_b[l, g]), linr_W[l, g], res[0], r1(last_gamma),
                     r1(last_beta), pred_W, r1(pred_b))
        return out
      nl, ng = (l, 1) if g == 0 else (l + 1, 0)
      y, o = _post_pre(res[g], o, aggp[0], aggp[1], c0, c1, linl_W[l, g],
                       r1(linl_b[l, g]), linr_W[l, g], r1(norm_gamma[nl, ng]),
                       r1(norm_beta[nl, ng]))
      res[g] = y


# speedup vs baseline: 5.7687x; 1.0837x over previous
import functools

import jax
import jax.numpy as jnp
from jax import lax
from jax.experimental import pallas as pl
from jax.experimental.pallas import tpu as pltpu
from jax.experimental.pallas import tpu_sc as plsc

N = 10000
E = 320000
D_IN = 128
HID = 128
HG = 64
OUT = 47
LAYERS = 4

NTILE = 16
NCORE = 2
NW = NTILE * NCORE
NPAD = 10112
STRIPE = NPAD // NTILE
CH = 128
NCHUNK = 80
EPW = CH * NCHUNK
EPAD = NW * EPW
NBUF = 8
LEAD = 4

@functools.cache
def _sc_mesh():
  return plsc.VectorSubcoreMesh(
      core_axis_name="c", subcore_axis_name="s", num_cores=NCORE,
      num_subcores=NTILE)


@functools.cache
def _make_sc_agg(D):
  @functools.partial(
      pl.kernel,
      out_type=jax.ShapeDtypeStruct((NCORE, NPAD, D), jnp.float32),
      mesh=_sc_mesh(),
      scratch_types=[
          pltpu.VMEM((NCHUNK, CH), jnp.int32),
          pltpu.VMEM((NCHUNK, CH), jnp.int32),
          [pltpu.VMEM((CH, D), jnp.float32)] * NBUF,
          [pltpu.SemaphoreType.DMA] * NBUF,
          [pltpu.SemaphoreType.DMA] * NBUF,
          pltpu.VMEM_SHARED((NPAD, D), jnp.float32),
      ],
      compiler_params=pltpu.CompilerParams(use_tc_tiling_on_sc=False),
  )
  def agg(table, src2d, dst2d, zeros, out, src_v, dst_v, rows, gsems, ssems,
          acc):
    c = lax.axis_index("c")
    s = lax.axis_index("s")
    w = c * NTILE + s
    pltpu.sync_copy(zeros.at[pl.ds(s * STRIPE, STRIPE)],
                    acc.at[pl.ds(s * STRIPE, STRIPE)])
    pltpu.sync_copy(src2d.at[w], src_v)
    pltpu.sync_copy(dst2d.at[w], dst_v)
    for i in range(LEAD):
      pltpu.async_copy(table.at[src_v.at[i]], rows[i], gsems[i])
    plsc.subcore_barrier()

    def body(g, carry):
      for b in range(NBUF):
        i = g * NBUF + b
        pltpu.make_async_copy(table.at[src_v.at[i]], rows[b],
                              gsems[b]).wait()
        pltpu.async_copy(rows[b], acc.at[dst_v.at[i]], ssems[b], add=True)
        j = i + LEAD
        bj = (b + LEAD) % NBUF

        @pl.when(j < NCHUNK)
        def _():
          @pl.when(j >= NBUF)
          def _():
            pltpu.make_async_copy(rows[bj], acc.at[dst_v.at[0]],
                                  ssems[bj]).wait()
          pltpu.async_copy(table.at[src_v.at[j]], rows[bj], gsems[bj])

      return carry

    lax.fori_loop(0, NCHUNK // NBUF, body, 0)
    for b in range(NBUF):
      pltpu.make_async_copy(rows[b], acc.at[dst_v.at[0]], ssems[b]).wait()
    plsc.subcore_barrier()
    pltpu.sync_copy(acc.at[pl.ds(s * STRIPE, STRIPE)],
                    out.at[c].at[pl.ds(s * STRIPE, STRIPE)])

  return agg


CD = 16


@functools.cache
def _make_sc_count():

  @functools.partial(
      pl.kernel,
      out_type=jax.ShapeDtypeStruct((NCORE, NPAD, CD), jnp.float32),
      mesh=_sc_mesh(),
      scratch_types=[
          pltpu.VMEM((NCHUNK, CH), jnp.int32),
          pltpu.VMEM((CH, CD), jnp.float32),
          pltpu.VMEM_SHARED((NPAD, CD), jnp.float32),
          pltpu.SemaphoreType.DMA,
      ],
      compiler_params=pltpu.CompilerParams(use_tc_tiling_on_sc=False),
  )
  def count(dst2d, ones, zeros, out, dst_v, ones_v, acc, ssem):
    c = lax.axis_index("c")
    s = lax.axis_index("s")
    w = c * NTILE + s
    pltpu.sync_copy(zeros.at[pl.ds(s * STRIPE, STRIPE)],
                    acc.at[pl.ds(s * STRIPE, STRIPE)])
    pltpu.sync_copy(dst2d.at[w], dst_v)
    pltpu.sync_copy(ones, ones_v)
    plsc.subcore_barrier()

    def body(i, carry):
      @pl.when(i >= NBUF)
      def _():
        pltpu.make_async_copy(ones_v, acc.at[dst_v.at[0]], ssem).wait()

      pltpu.async_copy(ones_v, acc.at[dst_v.at[i]], ssem, add=True)
      return carry

    lax.fori_loop(0, NCHUNK, body, 0)

    def drain(i, carry):
      pltpu.make_async_copy(ones_v, acc.at[dst_v.at[0]], ssem).wait()
      return carry

    lax.fori_loop(0, NBUF, drain, 0)
    plsc.subcore_barrier()
    pltpu.sync_copy(acc.at[pl.ds(s * STRIPE, STRIPE)],
                    out.at[c].at[pl.ds(s * STRIPE, STRIPE)])

  return count


R = 1000
GRID = N // R


def _ln_relu(x, g, b):
  mu = jnp.mean(x, axis=-1, keepdims=True)
  var = jnp.mean((x - mu) * (x - mu), axis=-1, keepdims=True)
  return jnp.maximum((x - mu) * lax.rsqrt(var + 1e-5) * g + b, 0.0)


def _enc_body(x_ref, w_ref, b_ref, g_ref, be_ref, x0_ref, x1_ref, a_ref):
  h = jnp.dot(x_ref[...], w_ref[...], preferred_element_type=jnp.float32)
  h = h + b_ref[...]
  x0_ref[...] = h[:, :HG]
  x1 = h[:, HG:]
  x1_ref[...] = x1
  a_ref[...] = _ln_relu(x1, g_ref[...], be_ref[...])


_full = lambda shape: pl.BlockSpec(shape, lambda i: (0,) * len(shape))
_rows = lambda shape: pl.BlockSpec(shape, lambda i: (i,) + (0,) * (len(shape) - 1))

_enc = pl.pallas_call(
    _enc_body,
    grid=(GRID,),
    in_specs=[_rows((R, D_IN)), _full((D_IN, HID)), _full((1, HID)),
              _full((1, HG)), _full((1, HG))],
    out_specs=[_rows((R, HG)), _rows((R, HG)), _rows((R, HG))],
    out_shape=[jax.ShapeDtypeStruct((N, HG), jnp.float32)] * 3,
)


def _mean_agg(a0, a1, c0, c1):
  cnt = jnp.maximum(c0[:, :1] + c1[:, :1], 1.0)
  return (a0 + a1) * (1.0 / cnt)


def _post_pre_body(y_ref, o_ref, a0_ref, a1_ref, c0_ref, c1_ref, wl_ref,
                   bl_ref, wr_ref, g_ref, b_ref, yo_ref, oo_ref):
  mean = _mean_agg(a0_ref[...], a1_ref[...], c0_ref[...], c1_ref[...])
  y = (y_ref[...] + bl_ref[...]
       + jnp.dot(mean, wl_ref[...], preferred_element_type=jnp.float32)
       + jnp.dot(o_ref[...], wr_ref[...], preferred_element_type=jnp.float32))
  yo_ref[...] = y
  oo_ref[...] = _ln_relu(y, g_ref[...], b_ref[...])


_post_pre = pl.pallas_call(
    _post_pre_body,
    grid=(GRID,),
    in_specs=[_rows((R, HG)), _rows((R, HG)), _rows((R, HG)), _rows((R, HG)),
              _rows((R, CD)), _rows((R, CD)), _full((HG, HG)), _full((1, HG)),
              _full((HG, HG)), _full((1, HG)), _full((1, HG))],
    out_specs=[_rows((R, HG)), _rows((R, HG))],
    out_shape=[jax.ShapeDtypeStruct((N, HG), jnp.float32)] * 2,
)


def _final_body(y_ref, o_ref, a0_ref, a1_ref, c0_ref, c1_ref, wl_ref, bl_ref,
                wr_ref, x0_ref, lg_ref, lb_ref, pw_ref, pb_ref, out_ref):
  mean = _mean_agg(a0_ref[...], a1_ref[...], c0_ref[...], c1_ref[...])
  y1 = (y_ref[...] + bl_ref[...]
        + jnp.dot(mean, wl_ref[...], preferred_element_type=jnp.float32)
        + jnp.dot(o_ref[...], wr_ref[...], preferred_element_type=jnp.float32))
  h = jnp.concatenate([x0_ref[...], y1], axis=-1)
  hn = _ln_relu(h, lg_ref[...], lb_ref[...])
  logits = jnp.dot(hn, pw_ref[...], preferred_element_type=jnp.float32)
  logits = logits + pb_ref[...]
  m = jnp.max(logits, axis=-1, keepdims=True)
  e = logits - m
  out_ref[...] = e - jnp.log(jnp.sum(jnp.exp(e), axis=-1, keepdims=True))


_final = pl.pallas_call(
    _final_body,
    grid=(GRID,),
    in_specs=[_rows((R, HG)), _rows((R, HG)), _rows((R, HG)), _rows((R, HG)),
              _rows((R, CD)), _rows((R, CD)), _full((HG, HG)), _full((1, HG)),
              _full((HG, HG)), _rows((R, HG)), _full((1, HID)),
              _full((1, HID)), _full((HID, OUT)), _full((1, OUT))],
    out_specs=_rows((R, OUT)),
    out_shape=jax.ShapeDtypeStruct((N, OUT), jnp.float32),
)


def kernel(x, edge_index, enc_W, enc_b, norm_gamma, norm_beta, linl_W, linl_b,
           linr_W, last_gamma, last_beta, pred_W, pred_b):
  src = edge_index[0]
  dst = edge_index[1]
  pad = EPAD - E
  src2d = jnp.concatenate(
      [src, jnp.zeros((pad,), jnp.int32)]).reshape(NW, NCHUNK, CH)
  dst2d = jnp.concatenate(
      [dst, jnp.full((pad,), N, jnp.int32)]).reshape(NW, NCHUNK, CH)
  zeros64 = jnp.zeros((NPAD, HG), jnp.float32)
  zeros16 = jnp.zeros((NPAD, CD), jnp.float32)
  ones16 = jnp.ones((CH, CD), jnp.float32)

  cntp = _make_sc_count()(dst2d, ones16, zeros16)
  c0 = cntp[0]
  c1 = cntp[1]

  r1 = lambda v: v.reshape(1, -1)
  x0, x1, o = _enc(x, enc_W, r1(enc_b), r1(norm_gamma[0, 0]),
                   r1(norm_beta[0, 0]))
  res = [x0, x1]
  for l in range(LAYERS):
    for g in range(2):
      aggp = _make_sc_agg(HG)(o, src2d, dst2d, zeros64)
      if (l, g) == (LAYERS - 1, 1):
        out = _final(res[1], o, aggp[0], aggp[1], c0, c1, linl_W[l, g],
                     r1(linl_b[l, g]), linr_W[l, g], res[0], r1(last_gamma),
                     r1(last_beta), pred_W, r1(pred_b))
        return out
      nl, ng = (l, 1) if g == 0 else (l + 1, 0)
      y, o = _post_pre(res[g], o, aggp[0], aggp[1], c0, c1, linl_W[l, g],
                       r1(linl_b[l, g]), linr_W[l, g], r1(norm_gamma[nl, ng]),
                       r1(norm_beta[nl, ng]))
      res[g] = y
